# Initial kernel scaffold; baseline (speedup 1.0000x reference)
import jax
import jax.numpy as jnp
from jax.experimental import pallas as pl


def kernel(x, edge_index, batch, params):
    raise NotImplementedError("write your pallas kernel here")


# speedup vs baseline: 9.1195x; 9.1195x over previous
import functools

import jax
import jax.numpy as jnp
from jax import lax
from jax.experimental import pallas as pl
from jax.experimental.pallas import tpu as pltpu
from jax.experimental.pallas import tpu_sc as plsc

N = 50000
E = 800000
NG = 512
H = 64
BN_EPS = 1e-5

NP = 50176
EP = 819200
RPT = NP // 16
EPT = EP // 16
MCH = 2048
MNC = EPT // MCH
MG = MCH // 16
SCH = 128
SNC = EPT // SCH
BR = 1024
GRID = NP // BR
PAD_SRC = N
PAD_DST = NP - 1

_f32 = jnp.float32


def _sc_chains_body(x_hbm, src_hbm, dst_hbm,
               a1_o, a2_o, a3_o, a4_o, agg0_o, deg_o,
               vals, acc, sidx, didx, res, tmp, red):
    sid = lax.axis_index("s")
    cid = lax.axis_index("c")
    zero16 = jnp.zeros((16,), _f32)
    one16 = jnp.ones((16,), _f32)

    def zero_acc():
        def zb(i, _):
            acc[pl.ds(i * 16, 16)] = zero16
            return 0
        lax.fori_loop(0, NP // 16, zb, 0)

    def one_pass(tab_hbm, out_hbm, gather, absval, addbase):
        zero_acc()
        if gather:
            pltpu.sync_copy(tab_hbm, vals)
            if absval:
                def ab(i, _):
                    vals[pl.ds(i * 16, 16)] = jnp.abs(vals[pl.ds(i * 16, 16)])
                    return 0
                lax.fori_loop(0, NP // 16, ab, 0)
        ebase = sid * EPT

        def chunk(j, _):
            if gather:
                pltpu.sync_copy(src_hbm.at[pl.ds(ebase + j * MCH, MCH)], sidx)
            pltpu.sync_copy(dst_hbm.at[pl.ds(ebase + j * MCH, MCH)], didx)

            def grp(g, _):
                d16 = didx[pl.ds(g * 16, 16)]
                if gather:
                    i16 = sidx[pl.ds(g * 16, 16)]
                    v = plsc.load_gather(vals, [i16])
                else:
                    v = one16
                plsc.addupdate_scatter(acc, [d16], v)
                return 0
            lax.fori_loop(0, MG, grp, 0)
            return 0
        lax.fori_loop(0, MNC, chunk, 0)

        rbase = sid * RPT

        def zr(i, _):
            res[pl.ds(i * 16, 16)] = zero16
            return 0
        lax.fori_loop(0, RPT // 16, zr, 0)
        for rnd in range(4):
            @pl.when((sid >= rnd * 4) & (sid < rnd * 4 + 4))
            def _():
                pltpu.sync_copy(acc, red.at[pl.ds((sid - rnd * 4) * NP, NP)])
            plsc.subcore_barrier()

            def redp(p, _):
                pltpu.sync_copy(red.at[pl.ds(p * NP + rbase, RPT)], tmp)

                def addg(i, _):
                    res[pl.ds(i * 16, 16)] = (res[pl.ds(i * 16, 16)]
                                              + tmp[pl.ds(i * 16, 16)])
                    return 0
                lax.fori_loop(0, RPT // 16, addg, 0)
                return 0
            lax.fori_loop(0, 4, redp, 0)
            plsc.subcore_barrier()
        if addbase:
            def addb(i, _):
                res[pl.ds(i * 16, 16)] = (res[pl.ds(i * 16, 16)]
                                          + vals[pl.ds(rbase + i * 16, 16)])
                return 0
            lax.fori_loop(0, RPT // 16, addb, 0)
        pltpu.sync_copy(res, out_hbm.at[pl.ds(rbase, RPT)])
        plsc.subcore_barrier()

    @pl.when(cid == 0)
    def _():
        one_pass(x_hbm, a1_o, True, True, True)
        one_pass(a1_o, a2_o, True, False, True)
        one_pass(a2_o, a3_o, True, False, True)
        one_pass(a3_o, a4_o, True, False, True)

    @pl.when(cid == 1)
    def _():
        one_pass(x_hbm, agg0_o, True, False, False)
        one_pass(x_hbm, deg_o, False, False, False)


def _sc_segsum_body(u0_hbm, u1_hbm, src_hbm, dst_hbm, zinit_hbm, o0, o1,
               sidx, didx, rows, acc, sem):
    sid = lax.axis_index("s")
    cid = lax.axis_index("c")
    rbase = sid * RPT
    pltpu.sync_copy(zinit_hbm, acc.at[pl.ds(rbase, RPT)])
    plsc.subcore_barrier()

    def body(u_hbm, o_hbm):
        ebase = sid * EPT

        def chunk(j, _):
            pltpu.sync_copy(src_hbm.at[pl.ds(ebase + j * SCH, SCH)], sidx)
            pltpu.sync_copy(dst_hbm.at[pl.ds(ebase + j * SCH, SCH)], didx)
            pltpu.async_copy(u_hbm.at[sidx], rows, sem).wait()
            pltpu.sync_copy(rows, acc.at[didx], add=True)
            return 0
        lax.fori_loop(0, SNC, chunk, 0)
        plsc.subcore_barrier()
        pltpu.sync_copy(acc.at[pl.ds(rbase, RPT)], o_hbm.at[pl.ds(rbase, RPT)])

    @pl.when(cid == 0)
    def _():
        body(u0_hbm, o0)

    @pl.when(cid == 1)
    def _():
        body(u1_hbm, o1)


@functools.cache
def _sc_mesh():
    return plsc.VectorSubcoreMesh(core_axis_name="c", subcore_axis_name="s",
                                  num_cores=2, num_subcores=16)


@functools.cache
def _sc_chains_kernel():
    return pl.kernel(
        _sc_chains_body,
        out_type=[jax.ShapeDtypeStruct((NP,), _f32)] * 6,
        mesh=_sc_mesh(),
        scratch_types=[
            pltpu.VMEM((NP,), _f32),
            pltpu.VMEM((NP,), _f32),
            pltpu.VMEM((MCH,), jnp.int32),
            pltpu.VMEM((MCH,), jnp.int32),
            pltpu.VMEM((RPT,), _f32),
            pltpu.VMEM((RPT,), _f32),
            pltpu.VMEM_SHARED((4 * NP,), _f32),
        ],
        compiler_params=pltpu.CompilerParams(needs_layout_passes=False),
    )


@functools.cache
def _sc_segsum_kernel():
    return pl.kernel(
        _sc_segsum_body,
        out_type=[jax.ShapeDtypeStruct((NP, 32), _f32)] * 2,
        mesh=_sc_mesh(),
        scratch_types=[
            pltpu.VMEM((SCH,), jnp.int32),
            pltpu.VMEM((SCH,), jnp.int32),
            pltpu.VMEM((SCH, 32), _f32),
            pltpu.VMEM_SHARED((NP, 32), _f32),
            pltpu.SemaphoreType.DMA,
        ],
        compiler_params=pltpu.CompilerParams(needs_layout_passes=False,
                                             use_tc_tiling_on_sc=False),
    )


def _sc_chains(xp, srcp, dstp):
    return _sc_chains_kernel()(xp, srcp, dstp)


def _sc_segsum(u0, u1, srcp, dstp, zinit):
    return _sc_segsum_kernel()(u0, u1, srcp, dstp, zinit)


def _wmask(b):
    rowid = b * BR + lax.broadcasted_iota(jnp.int32, (BR, 1), 0)
    return (rowid < N).astype(_f32)


def _acc_stats(b, st_ref, v):
    sums = jnp.sum(v, axis=0, keepdims=True)
    ssq = jnp.sum(v * v, axis=0, keepdims=True)
    blockst = jnp.concatenate([sums, ssq, jnp.zeros((6, H), _f32)], axis=0)

    @pl.when(b == 0)
    def _():
        st_ref[...] = blockst

    @pl.when(b != 0)
    def _():
        st_ref[...] = st_ref[...] + blockst


def _a0_body(x_ref, agg_ref, coef_ref, w2_ref, z2_ref, st_ref):
    b = pl.program_id(0)
    w1row = coef_ref[0:1, :]
    b1 = coef_ref[1:2, :]
    b2 = coef_ref[2:3, :]
    e1 = coef_ref[3:4, 0:1]
    z0 = e1 * x_ref[...] + agg_ref[...]
    z1 = jnp.maximum(z0 * w1row + b1, 0.0)
    z2 = jnp.maximum(jnp.dot(z1, w2_ref[...],
                             preferred_element_type=_f32) + b2, 0.0)
    z2_ref[...] = z2
    _acc_stats(b, st_ref, z2 * _wmask(b))


def _a_body(u0_ref, u1_ref, g0_ref, g1_ref, deg_ref, coef_ref, w1_ref,
            w2_ref, z2_ref, st_ref):
    b = pl.program_id(0)
    s = coef_ref[0:1, :]
    t = coef_ref[1:2, :]
    b1 = coef_ref[2:3, :]
    b2 = coef_ref[3:4, :]
    e1 = coef_ref[4:5, 0:1]
    u = jnp.concatenate([u0_ref[...], u1_ref[...]], axis=1)
    agg = jnp.concatenate([g0_ref[...], g1_ref[...]], axis=1)
    z0 = s * (e1 * u + agg) + t * (e1 + deg_ref[...])
    z1 = jnp.maximum(jnp.dot(z0, w1_ref[...],
                             preferred_element_type=_f32) + b1, 0.0)
    z2 = jnp.maximum(jnp.dot(z1, w2_ref[...],
                             preferred_element_type=_f32) + b2, 0.0)
    z2_ref[...] = z2
    _acc_stats(b, st_ref, z2 * _wmask(b))


def _b0_body(z2_ref, m_ref, coef_ref, o0_ref, o1_ref, st_ref):
    b = pl.program_id(0)
    cz1 = coef_ref[0:1, :]
    cz0 = coef_ref[1:2, :]
    zb = z2_ref[...] * cz1 + cz0
    v = jnp.where(zb > 0, zb, 0.01 * zb)
    v = v * (m_ref[...] > 0).astype(_f32) * _wmask(b)
    o0_ref[...] = v[:, :32]
    o1_ref[...] = v[:, 32:]
    _acc_stats(b, st_ref, v)


def _b_body(z2_ref, u0_ref, u1_ref, m_ref, coef_ref, o0_ref, o1_ref, st_ref):
    b = pl.program_id(0)
    cz1 = coef_ref[0:1, :]
    cz0 = coef_ref[1:2, :]
    s = coef_ref[2:3, :]
    t = coef_ref[3:4, :]
    zb = z2_ref[...] * cz1 + cz0
    lk = jnp.where(zb > 0, zb, 0.01 * zb)
    u = jnp.concatenate([u0_ref[...], u1_ref[...]], axis=1)
    v = (s * u + t + lk) * (m_ref[...] > 0).astype(_f32) * _wmask(b)
    o0_ref[...] = v[:, :32]
    o1_ref[...] = v[:, 32:]
    _acc_stats(b, st_ref, v)


def _h_body(u0_ref, u1_ref, m_ref, batch_ref, coef_ref, w1_ref, hc_ref,
            r_ref, bst_ref):
    b = pl.program_id(0)
    s = coef_ref[0:1, :]
    t = coef_ref[1:2, :]
    lin1b = hc_ref[0:1, :]
    lin2w = hc_ref[1:2, :]
    lin2b = hc_ref[2:3, 0:1]
    w = _wmask(b)
    m = (m_ref[...] > 0).astype(_f32) * w
    u = jnp.concatenate([u0_ref[...], u1_ref[...]], axis=1)
    h = s * u + t
    q = jnp.dot(h, w1_ref[...], preferred_element_type=_f32) + lin1b
    q = jnp.where(q > 0, q, 0.01 * q) * m
    r = jnp.sum(q * lin2w, axis=1, keepdims=True) + lin2b
    r = jnp.where(r > 0, r, 0.01 * r) * m
    r_ref[...] = r
    gid = lax.broadcasted_iota(jnp.int32, (1, NG), 1)
    onehot = (batch_ref[...] == gid) & (w > 0)
    cmax = jnp.max(jnp.where(onehot, r, -jnp.inf), axis=0, keepdims=True)
    cmin = jnp.min(jnp.where(onehot, r, jnp.inf), axis=0, keepdims=True)
    blockst = jnp.concatenate(
        [cmax, cmin, jnp.zeros((6, NG), _f32)], axis=0)

    @pl.when(b == 0)
    def _():
        bst_ref[...] = blockst

    @pl.when(b != 0)
    def _():
        prev = bst_ref[...]
        nmax = jnp.maximum(prev[0:1, :], cmax)
        nmin = jnp.minimum(prev[1:2, :], cmin)
        bst_ref[...] = jnp.concatenate([nmax, nmin, prev[2:, :]], axis=0)


def _r_body(r_ref, batch_ref, bst_ref, out_ref):
    gid = lax.broadcasted_iota(jnp.int32, (1, NG), 1)
    onehot = batch_ref[...] == gid
    bmax = jnp.sum(jnp.where(onehot, bst_ref[0:1, :], 0.0), axis=1,
                   keepdims=True)
    bmin = jnp.sum(jnp.where(onehot, bst_ref[1:2, :], 0.0), axis=1,
                   keepdims=True)
    r = r_ref[...]
    out_ref[...] = (r - bmin) / (bmax + 1e-6 - bmin)


def _col_spec():
    return pl.BlockSpec((BR, 1), lambda b: (b, 0))


def _half_spec():
    return pl.BlockSpec((BR, 32), lambda b: (b, 0))


def _full_spec():
    return pl.BlockSpec((BR, H), lambda b: (b, 0))


def _small_spec(shape):
    return pl.BlockSpec(shape, lambda b: (0, 0))


def _st_shape():
    return jax.ShapeDtypeStruct((8, H), _f32)


def _bn_coeffs(st, g, bb, eps):
    mu = st[0] / N
    var = st[1] / N - mu * mu
    c1 = g * lax.rsqrt(var + eps)
    return c1, bb - mu * c1


def _pack8(rows, width=H):
    out = []
    for r in rows:
        out.append(jnp.reshape(r, (1, width)))
    out.append(jnp.zeros((8 - len(rows), width), _f32))
    return jnp.concatenate(out, axis=0)


def kernel(x, edge_index, batch, params):
    x = x.astype(_f32)
    src = edge_index[0].astype(jnp.int32)
    dst = edge_index[1].astype(jnp.int32)
    srcp = jnp.concatenate(
        [src, jnp.full((EP - E,), PAD_SRC, jnp.int32)])
    dstp = jnp.concatenate(
        [dst, jnp.full((EP - E,), PAD_DST, jnp.int32)])
    xp = jnp.pad(x, (0, NP - N))
    batchp = jnp.pad(batch.astype(jnp.int32), (0, NP - N))[:, None]
    zinit = jnp.zeros((RPT, 32), _f32)

    a1, a2, a3, a4, agg0, deg = _sc_chains(xp, srcp, dstp)
    a1 = a1[:, None]
    a2 = a2[:, None]
    a3 = a3[:, None]
    a4 = a4[:, None]
    deg = deg[:, None]

    p = params["conv1"]
    coef = _pack8([p["W1"][0], p["b1"], p["b2"],
                   jnp.full((H,), 1.0 + p["eps"], _f32)])
    z2, st = pl.pallas_call(
        _a0_body,
        grid=(GRID,),
        in_specs=[_col_spec(), _col_spec(), _small_spec((8, H)),
                  _small_spec((H, H))],
        out_specs=[_full_spec(), _small_spec((8, H))],
        out_shape=[jax.ShapeDtypeStruct((NP, H), _f32), _st_shape()],
    )(xp[:, None], agg0[:, None], coef, p["W2"])
    cz1, cz0 = _bn_coeffs(st, p["bn_g"], p["bn_b"], BN_EPS)
    coefb = _pack8([cz1, cz0])
    u0, u1, st = pl.pallas_call(
        _b0_body,
        grid=(GRID,),
        in_specs=[_full_spec(), _col_spec(), _small_spec((8, H))],
        out_specs=[_half_spec(), _half_spec(), _small_spec((8, H))],
        out_shape=[jax.ShapeDtypeStruct((NP, 32), _f32),
                   jax.ShapeDtypeStruct((NP, 32), _f32), _st_shape()],
    )(z2, a1, coefb)
    s, t = _bn_coeffs(st, params["bn1"]["g"], params["bn1"]["b"], N * BN_EPS)

    masks = [a2, a3, a4]
    for j, (p, bn) in enumerate(zip(params["convs"], params["bns"])):
        g0, g1 = _sc_segsum(u0, u1, srcp, dstp, zinit)
        coefa = _pack8([s, t, p["b1"], p["b2"],
                        jnp.full((H,), 1.0 + p["eps"], _f32)])
        z2, st = pl.pallas_call(
            _a_body,
            grid=(GRID,),
            in_specs=[_half_spec(), _half_spec(), _half_spec(), _half_spec(),
                      _col_spec(), _small_spec((8, H)), _small_spec((H, H)),
                      _small_spec((H, H))],
            out_specs=[_full_spec(), _small_spec((8, H))],
            out_shape=[jax.ShapeDtypeStruct((NP, H), _f32), _st_shape()],
        )(u0, u1, g0, g1, deg, coefa, p["W1"], p["W2"])
        cz1, cz0 = _bn_coeffs(st, p["bn_g"], p["bn_b"], BN_EPS)
        coefb = _pack8([cz1, cz0, s, t])
        u0, u1, st = pl.pallas_call(
            _b_body,
            grid=(GRID,),
            in_specs=[_full_spec(), _half_spec(), _half_spec(), _col_spec(),
                      _small_spec((8, H))],
            out_specs=[_half_spec(), _half_spec(), _small_spec((8, H))],
            out_shape=[jax.ShapeDtypeStruct((NP, 32), _f32),
                       jax.ShapeDtypeStruct((NP, 32), _f32), _st_shape()],
        )(z2, u0, u1, masks[j], coefb)
        s, t = _bn_coeffs(st, bn["g"], bn["b"], N * BN_EPS)

    lw = params["lin1"]["W"]
    w1p = jnp.pad(lw, ((0, 0), (0, 128 - lw.shape[1])))
    hc = _pack8([jnp.pad(params["lin1"]["b"], (0, 120)),
                 jnp.pad(params["lin2"]["W"][:, 0], (0, 120)),
                 jnp.full((128,), params["lin2"]["b"][0], _f32)], width=128)
    coefh = _pack8([s, t])
    r, bst = pl.pallas_call(
        _h_body,
        grid=(GRID,),
        in_specs=[_half_spec(), _half_spec(), _col_spec(), _col_spec(),
                  _small_spec((8, H)), _small_spec((H, 128)),
                  _small_spec((8, 128))],
        out_specs=[_col_spec(), _small_spec((8, NG))],
        out_shape=[jax.ShapeDtypeStruct((NP, 1), _f32),
                   jax.ShapeDtypeStruct((8, NG), _f32)],
    )(u0, u1, a4, batchp, coefh, w1p, hc)
    out = pl.pallas_call(
        _r_body,
        grid=(GRID,),
        in_specs=[_col_spec(), _col_spec(), _small_spec((8, NG))],
        out_specs=_col_spec(),
        out_shape=jax.ShapeDtypeStruct((NP, 1), _f32),
    )(r, batchp, bst)
    return out[:N]


# speedup vs baseline: 11.9022x; 1.3051x over previous
import functools

import jax
import jax.numpy as jnp
from jax import lax
from jax.experimental import pallas as pl
from jax.experimental.pallas import tpu as pltpu
from jax.experimental.pallas import tpu_sc as plsc

N = 50000
E = 800000
NG = 512
H = 64
BN_EPS = 1e-5

NP = 50176
EP = 819200
RPT = NP // 16
EPT = EP // 16
MCH = 2048
MNC = EPT // MCH
MG = MCH // 16
SCH = 128
SB = 3
SLAB = SB * SCH
NSLAB = 136
EPTS = NSLAB * SLAB
EPS = 16 * EPTS
SROWS = EPTS // SCH
BR = 1024
GRID = NP // BR
PAD_SRC = N
PAD_DST = NP - 1

_f32 = jnp.float32


def _sc_chains_body(x_hbm, src_hbm, dst_hbm, z1d_hbm,
               a1_o, a2_o, a3_o, a4_o, agg0_o, deg_o,
               vals, acc, sidx, didx, res, tmp, red):
    sid = lax.axis_index("s")
    cid = lax.axis_index("c")
    one16 = jnp.ones((16,), _f32)

    def one_pass(tab_hbm, out_hbm, gather, absval, addbase):
        pltpu.sync_copy(z1d_hbm, acc)
        if gather:
            pltpu.sync_copy(tab_hbm, vals)
            if absval:
                def ab(i, _):
                    for q in range(4):
                        o = i * 64 + q * 16
                        vals[pl.ds(o, 16)] = jnp.abs(vals[pl.ds(o, 16)])
                    return 0
                lax.fori_loop(0, NP // 64, ab, 0)
        ebase = sid * EPT

        def chunk(j, _):
            if gather:
                pltpu.sync_copy(src_hbm.at[pl.ds(ebase + j * MCH, MCH)], sidx)
            pltpu.sync_copy(dst_hbm.at[pl.ds(ebase + j * MCH, MCH)], didx)

            def grp(g, _):
                for q in range(8):
                    d16 = didx[pl.ds(g * 128 + q * 16, 16)]
                    if gather:
                        i16 = sidx[pl.ds(g * 128 + q * 16, 16)]
                        v = plsc.load_gather(vals, [i16])
                    else:
                        v = one16
                    plsc.addupdate_scatter(acc, [d16], v)
                return 0
            lax.fori_loop(0, MG // 8, grp, 0)
            return 0
        lax.fori_loop(0, MNC, chunk, 0)

        rbase = sid * RPT

        pltpu.sync_copy(z1d_hbm.at[pl.ds(0, RPT)], res)
        for rnd in range(4):
            @pl.when((sid >= rnd * 4) & (sid < rnd * 4 + 4))
            def _():
                pltpu.sync_copy(acc, red.at[pl.ds((sid - rnd * 4) * NP, NP)])
            plsc.subcore_barrier()

            def redp(p, _):
                pltpu.sync_copy(red.at[pl.ds(p * NP + rbase, RPT)], tmp)

                def addg(i, _):
                    for q in range(4):
                        o = i * 64 + q * 16
                        res[pl.ds(o, 16)] = (res[pl.ds(o, 16)]
                                             + tmp[pl.ds(o, 16)])
                    return 0
                lax.fori_loop(0, RPT // 64, addg, 0)
                return 0
            lax.fori_loop(0, 4, redp, 0)
            plsc.subcore_barrier()
        if addbase:
            def addb(i, _):
                for q in range(4):
                    o = i * 64 + q * 16
                    res[pl.ds(o, 16)] = (res[pl.ds(o, 16)]
                                         + vals[pl.ds(rbase + o, 16)])
                return 0
            lax.fori_loop(0, RPT // 64, addb, 0)
        pltpu.sync_copy(res, out_hbm.at[pl.ds(rbase, RPT)])
        plsc.subcore_barrier()

    @pl.when(cid == 0)
    def _():
        one_pass(x_hbm, a1_o, True, True, True)
        one_pass(a1_o, a2_o, True, False, True)
        one_pass(a2_o, a3_o, True, False, True)
        one_pass(a3_o, a4_o, True, False, True)

    @pl.when(cid == 1)
    def _():
        one_pass(x_hbm, agg0_o, True, False, False)
        one_pass(x_hbm, deg_o, False, False, False)


def _sc_segsum_body(u0_hbm, u1_hbm, src2_hbm, dst2_hbm, zinit_hbm, o0, o1,
               s0, d0, s1, d1, rows0, rows1, acc,
               gsem0, gsem1, ssem0, ssem1):
    sid = lax.axis_index("s")
    cid = lax.axis_index("c")
    rbase = sid * RPT
    pltpu.sync_copy(zinit_hbm, acc.at[pl.ds(rbase, RPT)])
    plsc.subcore_barrier()
    bufsets = ((s0, d0, rows0, gsem0, ssem0), (s1, d1, rows1, gsem1, ssem1))

    def body(u_hbm, o_hbm):
        irb = sid * SROWS

        def load_idx(i, bufs):
            pltpu.sync_copy(src2_hbm.at[pl.ds(irb + i * SB, SB)], bufs[0])
            pltpu.sync_copy(dst2_hbm.at[pl.ds(irb + i * SB, SB)], bufs[1])

        def fire_gathers(bufs):
            sbuf, _, rows, gsem, _ = bufs
            for k in range(SB):
                pltpu.async_copy(u_hbm.at[sbuf.at[k]],
                                 rows.at[pl.ds(k * SCH, SCH)], gsem)

        def wait_gathers(bufs):
            sbuf, _, rows, gsem, _ = bufs
            for k in range(SB):
                pltpu.make_async_copy(u_hbm.at[sbuf.at[k]],
                                      rows.at[pl.ds(k * SCH, SCH)],
                                      gsem).wait()

        def fire_scatters(bufs):
            _, dbuf, rows, _, ssem = bufs
            for k in range(SB):
                pltpu.async_copy(rows.at[pl.ds(k * SCH, SCH)],
                                 acc.at[dbuf.at[k]], ssem, add=True)

        def drain_scatters(bufs):
            _, dbuf, rows, _, ssem = bufs
            for k in range(SB):
                pltpu.make_async_copy(rows.at[pl.ds(k * SCH, SCH)],
                                      acc.at[dbuf.at[k]], ssem).wait()

        load_idx(0, bufsets[0])
        fire_gathers(bufsets[0])

        def step(i, cur, oth):
            @pl.when(i >= 1)
            def _():
                drain_scatters(oth)

            @pl.when(i + 1 < NSLAB)
            def _():
                load_idx(i + 1, oth)
                fire_gathers(oth)
            wait_gathers(cur)
            fire_scatters(cur)

        def it(i, _):
            @pl.when(i % 2 == 0)
            def _():
                step(i, bufsets[0], bufsets[1])

            @pl.when(i % 2 == 1)
            def _():
                step(i, bufsets[1], bufsets[0])
            return 0
        lax.fori_loop(0, NSLAB, it, 0)
        drain_scatters(bufsets[(NSLAB - 1) % 2])
        plsc.subcore_barrier()
        pltpu.sync_copy(acc.at[pl.ds(rbase, RPT)], o_hbm.at[pl.ds(rbase, RPT)])

    @pl.when(cid == 0)
    def _():
        body(u0_hbm, o0)

    @pl.when(cid == 1)
    def _():
        body(u1_hbm, o1)


@functools.cache
def _sc_mesh():
    return plsc.VectorSubcoreMesh(core_axis_name="c", subcore_axis_name="s",
                                  num_cores=2, num_subcores=16)


@functools.cache
def _sc_chains_kernel():
    return pl.kernel(
        _sc_chains_body,
        out_type=[jax.ShapeDtypeStruct((NP,), _f32)] * 6,
        mesh=_sc_mesh(),
        scratch_types=[
            pltpu.VMEM((NP,), _f32),
            pltpu.VMEM((NP,), _f32),
            pltpu.VMEM((MCH,), jnp.int32),
            pltpu.VMEM((MCH,), jnp.int32),
            pltpu.VMEM((RPT,), _f32),
            pltpu.VMEM((RPT,), _f32),
            pltpu.VMEM_SHARED((4 * NP,), _f32),
        ],
        compiler_params=pltpu.CompilerParams(needs_layout_passes=False),
    )


@functools.cache
def _sc_segsum_kernel():
    return pl.kernel(
        _sc_segsum_body,
        out_type=[jax.ShapeDtypeStruct((NP, 32), _f32)] * 2,
        mesh=_sc_mesh(),
        scratch_types=[
            pltpu.VMEM((SB, SCH), jnp.int32),
            pltpu.VMEM((SB, SCH), jnp.int32),
            pltpu.VMEM((SB, SCH), jnp.int32),
            pltpu.VMEM((SB, SCH), jnp.int32),
            pltpu.VMEM((SLAB, 32), _f32),
            pltpu.VMEM((SLAB, 32), _f32),
            pltpu.VMEM_SHARED((NP, 32), _f32),
            pltpu.SemaphoreType.DMA,
            pltpu.SemaphoreType.DMA,
            pltpu.SemaphoreType.DMA,
            pltpu.SemaphoreType.DMA,
        ],
        compiler_params=pltpu.CompilerParams(needs_layout_passes=False,
                                             use_tc_tiling_on_sc=False),
    )


def _sc_chains(xp, srcp, dstp, z1d):
    return _sc_chains_kernel()(xp, srcp, dstp, z1d)


def _sc_segsum(u0, u1, src2, dst2, zinit):
    return _sc_segsum_kernel()(u0, u1, src2, dst2, zinit)


def _wmask(b):
    rowid = b * BR + lax.broadcasted_iota(jnp.int32, (BR, 1), 0)
    return (rowid < N).astype(_f32)


def _acc_stats(b, st_ref, v):
    sums = jnp.sum(v, axis=0, keepdims=True)
    ssq = jnp.sum(v * v, axis=0, keepdims=True)
    blockst = jnp.concatenate([sums, ssq, jnp.zeros((6, H), _f32)], axis=0)

    @pl.when(b == 0)
    def _():
        st_ref[...] = blockst

    @pl.when(b != 0)
    def _():
        st_ref[...] = st_ref[...] + blockst


def _a0_body(x_ref, agg_ref, coef_ref, w2_ref, z2_ref, st_ref):
    b = pl.program_id(0)
    w1row = coef_ref[0:1, :]
    b1 = coef_ref[1:2, :]
    b2 = coef_ref[2:3, :]
    e1 = coef_ref[3:4, 0:1]
    z0 = e1 * x_ref[...] + agg_ref[...]
    z1 = jnp.maximum(z0 * w1row + b1, 0.0)
    z2 = jnp.maximum(jnp.dot(z1, w2_ref[...],
                             preferred_element_type=_f32) + b2, 0.0)
    z2_ref[...] = z2
    _acc_stats(b, st_ref, z2 * _wmask(b))


def _a_body(u0_ref, u1_ref, g0_ref, g1_ref, deg_ref, coef_ref, w1_ref,
            w2_ref, z2_ref, st_ref):
    b = pl.program_id(0)
    s = coef_ref[0:1, :]
    t = coef_ref[1:2, :]
    b1 = coef_ref[2:3, :]
    b2 = coef_ref[3:4, :]
    e1 = coef_ref[4:5, 0:1]
    u = jnp.concatenate([u0_ref[...], u1_ref[...]], axis=1)
    agg = jnp.concatenate([g0_ref[...], g1_ref[...]], axis=1)
    z0 = s * (e1 * u + agg) + t * (e1 + deg_ref[...])
    z1 = jnp.maximum(jnp.dot(z0, w1_ref[...],
                             preferred_element_type=_f32) + b1, 0.0)
    z2 = jnp.maximum(jnp.dot(z1, w2_ref[...],
                             preferred_element_type=_f32) + b2, 0.0)
    z2_ref[...] = z2
    _acc_stats(b, st_ref, z2 * _wmask(b))


def _b0_body(z2_ref, m_ref, coef_ref, o0_ref, o1_ref, st_ref):
    b = pl.program_id(0)
    cz1 = coef_ref[0:1, :]
    cz0 = coef_ref[1:2, :]
    zb = z2_ref[...] * cz1 + cz0
    v = jnp.where(zb > 0, zb, 0.01 * zb)
    v = v * (m_ref[...] > 0).astype(_f32) * _wmask(b)
    o0_ref[...] = v[:, :32]
    o1_ref[...] = v[:, 32:]
    _acc_stats(b, st_ref, v)


def _b_body(z2_ref, u0_ref, u1_ref, m_ref, coef_ref, o0_ref, o1_ref, st_ref):
    b = pl.program_id(0)
    cz1 = coef_ref[0:1, :]
    cz0 = coef_ref[1:2, :]
    s = coef_ref[2:3, :]
    t = coef_ref[3:4, :]
    zb = z2_ref[...] * cz1 + cz0
    lk = jnp.where(zb > 0, zb, 0.01 * zb)
    u = jnp.concatenate([u0_ref[...], u1_ref[...]], axis=1)
    v = (s * u + t + lk) * (m_ref[...] > 0).astype(_f32) * _wmask(b)
    o0_ref[...] = v[:, :32]
    o1_ref[...] = v[:, 32:]
    _acc_stats(b, st_ref, v)


def _h_body(u0_ref, u1_ref, m_ref, batch_ref, coef_ref, w1_ref, hc_ref,
            r_ref, bst_ref):
    b = pl.program_id(0)
    s = coef_ref[0:1, :]
    t = coef_ref[1:2, :]
    lin1b = hc_ref[0:1, :]
    lin2w = hc_ref[1:2, :]
    lin2b = hc_ref[2:3, 0:1]
    w = _wmask(b)
    m = (m_ref[...] > 0).astype(_f32) * w
    u = jnp.concatenate([u0_ref[...], u1_ref[...]], axis=1)
    h = s * u + t
    q = jnp.dot(h, w1_ref[...], preferred_element_type=_f32) + lin1b
    q = jnp.where(q > 0, q, 0.01 * q) * m
    r = jnp.sum(q * lin2w, axis=1, keepdims=True) + lin2b
    r = jnp.where(r > 0, r, 0.01 * r) * m
    r_ref[...] = r
    gid = lax.broadcasted_iota(jnp.int32, (1, NG), 1)
    onehot = (batch_ref[...] == gid) & (w > 0)
    cmax = jnp.max(jnp.where(onehot, r, -jnp.inf), axis=0, keepdims=True)
    cmin = jnp.min(jnp.where(onehot, r, jnp.inf), axis=0, keepdims=True)
    blockst = jnp.concatenate(
        [cmax, cmin, jnp.zeros((6, NG), _f32)], axis=0)

    @pl.when(b == 0)
    def _():
        bst_ref[...] = blockst

    @pl.when(b != 0)
    def _():
        prev = bst_ref[...]
        nmax = jnp.maximum(prev[0:1, :], cmax)
        nmin = jnp.minimum(prev[1:2, :], cmin)
        bst_ref[...] = jnp.concatenate([nmax, nmin, prev[2:, :]], axis=0)


def _r_body(r_ref, batch_ref, bst_ref, out_ref):
    gid = lax.broadcasted_iota(jnp.int32, (1, NG), 1)
    onehot = batch_ref[...] == gid
    bmax = jnp.sum(jnp.where(onehot, bst_ref[0:1, :], 0.0), axis=1,
                   keepdims=True)
    bmin = jnp.sum(jnp.where(onehot, bst_ref[1:2, :], 0.0), axis=1,
                   keepdims=True)
    r = r_ref[...]
    out_ref[...] = (r - bmin) / (bmax + 1e-6 - bmin)


def _col_spec():
    return pl.BlockSpec((BR, 1), lambda b: (b, 0))


def _half_spec():
    return pl.BlockSpec((BR, 32), lambda b: (b, 0))


def _full_spec():
    return pl.BlockSpec((BR, H), lambda b: (b, 0))


def _small_spec(shape):
    return pl.BlockSpec(shape, lambda b: (0, 0))


def _st_shape():
    return jax.ShapeDtypeStruct((8, H), _f32)


def _bn_coeffs(st, g, bb, eps):
    mu = st[0] / N
    var = st[1] / N - mu * mu
    c1 = g * lax.rsqrt(var + eps)
    return c1, bb - mu * c1


def _pack8(rows, width=H):
    out = []
    for r in rows:
        out.append(jnp.reshape(r, (1, width)))
    out.append(jnp.zeros((8 - len(rows), width), _f32))
    return jnp.concatenate(out, axis=0)


def kernel(x, edge_index, batch, params):
    x = x.astype(_f32)
    src = edge_index[0].astype(jnp.int32)
    dst = edge_index[1].astype(jnp.int32)
    srcp = jnp.concatenate(
        [src, jnp.full((EP - E,), PAD_SRC, jnp.int32)])
    dstp = jnp.concatenate(
        [dst, jnp.full((EP - E,), PAD_DST, jnp.int32)])
    src2 = jnp.concatenate(
        [src, jnp.full((EPS - E,), PAD_SRC, jnp.int32)]).reshape(
            EPS // SCH, SCH)
    dst2 = jnp.concatenate(
        [dst, jnp.full((EPS - E,), PAD_DST, jnp.int32)]).reshape(
            EPS // SCH, SCH)
    xp = jnp.pad(x, (0, NP - N))
    z1d = jnp.zeros((NP,), _f32)
    batchp = jnp.pad(batch.astype(jnp.int32), (0, NP - N))[:, None]
    zinit = jnp.zeros((RPT, 32), _f32)

    a1, a2, a3, a4, agg0, deg = _sc_chains(xp, srcp, dstp, z1d)
    a1 = a1[:, None]
    a2 = a2[:, None]
    a3 = a3[:, None]
    a4 = a4[:, None]
    deg = deg[:, None]

    p = params["conv1"]
    coef = _pack8([p["W1"][0], p["b1"], p["b2"],
                   jnp.full((H,), 1.0 + p["eps"], _f32)])
    z2, st = pl.pallas_call(
        _a0_body,
        grid=(GRID,),
        in_specs=[_col_spec(), _col_spec(), _small_spec((8, H)),
                  _small_spec((H, H))],
        out_specs=[_full_spec(), _small_spec((8, H))],
        out_shape=[jax.ShapeDtypeStruct((NP, H), _f32), _st_shape()],
    )(xp[:, None], agg0[:, None], coef, p["W2"])
    cz1, cz0 = _bn_coeffs(st, p["bn_g"], p["bn_b"], BN_EPS)
    coefb = _pack8([cz1, cz0])
    u0, u1, st = pl.pallas_call(
        _b0_body,
        grid=(GRID,),
        in_specs=[_full_spec(), _col_spec(), _small_spec((8, H))],
        out_specs=[_half_spec(), _half_spec(), _small_spec((8, H))],
        out_shape=[jax.ShapeDtypeStruct((NP, 32), _f32),
                   jax.ShapeDtypeStruct((NP, 32), _f32), _st_shape()],
    )(z2, a1, coefb)
    s, t = _bn_coeffs(st, params["bn1"]["g"], params["bn1"]["b"], N * BN_EPS)

    masks = [a2, a3, a4]
    for j, (p, bn) in enumerate(zip(params["convs"], params["bns"])):
        g0, g1 = _sc_segsum(u0, u1, src2, dst2, zinit)
        coefa = _pack8([s, t, p["b1"], p["b2"],
                        jnp.full((H,), 1.0 + p["eps"], _f32)])
        z2, st = pl.pallas_call(
            _a_body,
            grid=(GRID,),
            in_specs=[_half_spec(), _half_spec(), _half_spec(), _half_spec(),
                      _col_spec(), _small_spec((8, H)), _small_spec((H, H)),
                      _small_spec((H, H))],
            out_specs=[_full_spec(), _small_spec((8, H))],
            out_shape=[jax.ShapeDtypeStruct((NP, H), _f32), _st_shape()],
        )(u0, u1, g0, g1, deg, coefa, p["W1"], p["W2"])
        cz1, cz0 = _bn_coeffs(st, p["bn_g"], p["bn_b"], BN_EPS)
        coefb = _pack8([cz1, cz0, s, t])
        u0, u1, st = pl.pallas_call(
            _b_body,
            grid=(GRID,),
            in_specs=[_full_spec(), _half_spec(), _half_spec(), _col_spec(),
                      _small_spec((8, H))],
            out_specs=[_half_spec(), _half_spec(), _small_spec((8, H))],
            out_shape=[jax.ShapeDtypeStruct((NP, 32), _f32),
                       jax.ShapeDtypeStruct((NP, 32), _f32), _st_shape()],
        )(z2, u0, u1, masks[j], coefb)
        s, t = _bn_coeffs(st, bn["g"], bn["b"], N * BN_EPS)

    lw = params["lin1"]["W"]
    w1p = jnp.pad(lw, ((0, 0), (0, 128 - lw.shape[1])))
    hc = _pack8([jnp.pad(params["lin1"]["b"], (0, 120)),
                 jnp.pad(params["lin2"]["W"][:, 0], (0, 120)),
                 jnp.full((128,), params["lin2"]["b"][0], _f32)], width=128)
    coefh = _pack8([s, t])
    r, bst = pl.pallas_call(
        _h_body,
        grid=(GRID,),
        in_specs=[_half_spec(), _half_spec(), _col_spec(), _col_spec(),
                  _small_spec((8, H)), _small_spec((H, 128)),
                  _small_spec((8, 128))],
        out_specs=[_col_spec(), _small_spec((8, NG))],
        out_shape=[jax.ShapeDtypeStruct((NP, 1), _f32),
                   jax.ShapeDtypeStruct((8, NG), _f32)],
    )(u0, u1, a4, batchp, coefh, w1p, hc)
    out = pl.pallas_call(
        _r_body,
        grid=(GRID,),
        in_specs=[_col_spec(), _col_spec(), _small_spec((8, NG))],
        out_specs=_col_spec(),
        out_shape=jax.ShapeDtypeStruct((NP, 1), _f32),
    )(r, batchp, bst)
    return out[:N]
